# baseline (speedup 1.0000x reference)
import jax
import jax.numpy as jnp
from jax import lax
from jax.experimental import pallas as pl
from jax.experimental.pallas import tpu as pltpu
from jax.experimental.pallas import tpu_sc as plsc

N = 10000
E = 160000
NHOP = 40000
RN = 500
ALPHA = 0.2

NC = 2
NS = 16
NW = NC * NS

NPAD = 10240
RPAD = 512
EPAD = 163840
HPAD = 40960
CH = 32
D = 128
DG = 144
ROWS_PER_TILE = NPAD // NS
ET_N, EC_N = EPAD // NW, EPAD // NC
ET_H, EC_H = HPAD // NW, HPAD // NC
BT = 4096 // NW


def _leakyexp(z):
    return jnp.exp(-jnp.where(z >= 0, z, ALPHA * z))


def _splat(vec, idx16):
    dnums = lax.GatherDimensionNumbers(
        offset_dims=(), collapsed_slice_dims=(0,), start_index_map=(0,))
    return lax.gather(vec, idx16[:, None], dnums, (1,),
                      mode=lax.GatherScatterMode.PROMISE_IN_BOUNDS)


def _zero_rows(buf, width, nrows):
    def body(i, c):
        for d in range(width // 16):
            buf[i, pl.ds(d * 16, 16)] = jnp.zeros((16,), jnp.float32)
        return c
    lax.fori_loop(0, nrows, body, 0)


BLK = 2


def _att_body(nheads,
              dst_n, src_n, rt_n, dst_h, src_h, ra_h, rb_h, bidx_h,
              psrc_hbm, prel_hbm, tdst_hbm,
              out_n, out_w,
              accum_n, accum_w,
              dstblk0, dstblk1, srcblk0, srcblk1, rablk0, rablk1,
              rbblk0, rbblk1, dst_v0, dst_v1,
              rows_s0, rows_s1, rows_a0, rows_a1, rows_b0, rows_b1,
              dscal0, dscal1,
              out_rows0, out_rows1, w_rows0, w_rows1,
              sem0, sem1, semi0, semi1, semw0, semw1):
    cid = lax.axis_index("c")
    sid = lax.axis_index("s")
    lane = lax.iota(jnp.int32, 16)
    zeros16 = jnp.zeros((16,), jnp.int32)
    dstblk = [dstblk0, dstblk1]
    srcblk = [srcblk0, srcblk1]
    rablk = [rablk0, rablk1]
    rbblk = [rbblk0, rbblk1]
    dst_v = [dst_v0, dst_v1]
    rows_s = [rows_s0, rows_s1]
    rows_a = [rows_a0, rows_a1]
    rows_b = [rows_b0, rows_b1]
    dscal = [dscal0, dscal1]
    out_rows = [out_rows0, out_rows1]
    w_rows = [w_rows0, w_rows1]
    sem = [sem0, sem1]
    semi = [semi0, semi1]
    semw = [semw0, semw1]
    SPAN = BLK * CH

    _zero_rows(out_rows[0], D, CH)
    _zero_rows(w_rows[0], 16, CH)
    for j in range(ROWS_PER_TILE // CH):
        st = sid * ROWS_PER_TILE + j * CH
        pltpu.sync_copy(out_rows[0], accum_n.at[pl.ds(st, CH)])
        pltpu.sync_copy(w_rows[0], accum_w.at[pl.ds(st, CH)])
    plsc.subcore_barrier()

    if bidx_h is not None:
        def mrow(i, c):
            w_rows[0][i, pl.ds(0, 16)] = jnp.where(lane == 1, 1.0, 0.0)
            return c
        lax.fori_loop(0, CH, mrow, 0)
        wid = cid * NS + sid
        for j in range(BT // CH):
            pltpu.sync_copy(bidx_h.at[pl.ds(wid * BT + j * CH, CH)], dst_v[0])
            pltpu.sync_copy(w_rows[0], accum_w.at[dst_v[0]], add=True)

    def idx_arrays(nhop):
        return ([dst_h, src_h, ra_h, rb_h] if nhop
                else [dst_n, src_n, rt_n])

    def idx_bufs(p, nhop):
        return ([dstblk[p], srcblk[p], rablk[p], rbblk[p]] if nhop
                else [dstblk[p], srcblk[p], rablk[p]])

    def issue_idx(p, brow, nhop):
        for arr, buf in zip(idx_arrays(nhop), idx_bufs(p, nhop)):
            pltpu.async_copy(arr.at[pl.ds(brow, BLK)], buf, semi[p])

    def drain_idx(p, brow, nhop):
        for arr, buf in zip(idx_arrays(nhop), idx_bufs(p, nhop)):
            pltpu.make_async_copy(arr.at[pl.ds(brow, BLK)], buf,
                                  semi[p]).wait()

    def gather_list(b, p, k, nhop):
        pairs = [
            (psrc_hbm.at[srcblk[p].at[k]], rows_s[b]),
            (prel_hbm.at[rablk[p].at[k]], rows_a[b]),
            (tdst_hbm.at[dstblk[p].at[k]], dscal[b]),
        ]
        if nhop:
            pairs += [(prel_hbm.at[rbblk[p].at[k]], rows_b[b])]
        return pairs

    def issue(b, p, k, nhop):
        for s, d in gather_list(b, p, k, nhop):
            pltpu.async_copy(s, d, sem[b])

    def drain(b, p, k, nhop):
        for s, d in gather_list(b, p, k, nhop):
            pltpu.make_async_copy(s, d, sem[b]).wait()

    def drain_scatter(b):
        pltpu.make_async_copy(out_rows[b], accum_n.at[dst_v[b]],
                              semw[b]).wait()
        pltpu.make_async_copy(w_rows[b], accum_w.at[dst_v[b]],
                              semw[b]).wait()

    def compute(b, p, k, nhop):
        for g in range(CH // 16):
            dst_v[b][pl.ds(g * 16, 16)] = dstblk[p][k, pl.ds(g * 16, 16)]

        def eb(e, c):
            sl = pl.ds(0, 16)
            slz = pl.ds(D, 16)
            z16 = dscal[b][e, sl] + rows_s[b][e, slz] + rows_a[b][e, slz]
            if nhop:
                z16 = z16 + rows_b[b][e, slz]
            w16 = _leakyexp(z16)
            w_rows[b][e, sl] = jnp.where(lane < nheads, w16, 0.0)
            w0 = _splat(w16, zeros16)
            if nheads == 2:
                w1 = _splat(w16, zeros16 + 1)
            for d in range(8):
                sld = pl.ds(d * 16, 16)
                v = rows_s[b][e, sld] + rows_a[b][e, sld]
                if nhop:
                    v = v + rows_b[b][e, sld]
                if nheads == 2:
                    out_rows[b][e, sld] = v * (w0 if d < 4 else w1)
                else:
                    out_rows[b][e, sld] = v * w0
        plsc.parallel_loop(0, CH, unroll=6)(lambda e: eb(e, 0))
        pltpu.async_copy(out_rows[b], accum_n.at[dst_v[b]], semw[b], add=True)
        pltpu.async_copy(w_rows[b], accum_w.at[dst_v[b]], semw[b], add=True)

    def edge_loop(nchunks, cbase, nhop, first_loop):
        nblocks = nchunks // BLK
        crow = cbase // CH
        issue_idx(0, crow, nhop)
        drain_idx(0, crow, nhop)
        issue(0, 0, 0, nhop)
        issue_idx(1, crow + BLK, nhop)

        def block_pair(i2, c):
            for parity in range(2):
                ib = i2 * 2 + parity
                brow = crow + ib * BLK
                for k in range(BLK):
                    b = k % 2
                    drain(b, parity, k, nhop)
                    if k + 1 < BLK:
                        issue(1 - b, parity, k + 1, nhop)
                    else:
                        @pl.when(ib + 1 < nblocks)
                        def _():
                            drain_idx(1 - parity, brow + BLK, nhop)
                            issue(1 - b, 1 - parity, 0, nhop)
                    if first_loop:
                        @pl.when(ib * BLK + k >= 2)
                        def _():
                            drain_scatter(b)
                    else:
                        drain_scatter(b)
                    compute(b, parity, k, nhop)

                @pl.when(ib + 2 < nblocks)
                def _():
                    issue_idx(parity, brow + 2 * BLK, nhop)
            return c
        lax.fori_loop(0, nblocks // 2, block_pair, 0)

    edge_loop(ET_N // CH, cid * EC_N + sid * ET_N, False, True)
    edge_loop(ET_H // CH, cid * EC_H + sid * ET_H, True, False)
    drain_scatter(0)
    drain_scatter(1)

    plsc.subcore_barrier()
    for j in range(ROWS_PER_TILE // CH):
        st = sid * ROWS_PER_TILE + j * CH
        pltpu.sync_copy(accum_n.at[pl.ds(st, CH)], out_rows[0])
        pltpu.sync_copy(out_rows[0], out_n.at[pl.ds(cid * NPAD + st, CH)])
        pltpu.sync_copy(accum_w.at[pl.ds(st, CH)], w_rows[0])
        pltpu.sync_copy(w_rows[0], out_w.at[pl.ds(cid * NPAD + st, CH)])


def _att1_body(dst_n, src_n, rt_n, dst_h, src_h, ra_h, rb_h,
               psrc_hbm, prel_hbm, tdst_hbm,
               out_n, out_w, *rest):
    _att_body(2, dst_n, src_n, rt_n, dst_h, src_h, ra_h, rb_h, None,
              psrc_hbm, prel_hbm, tdst_hbm,
              out_n, out_w, *rest)


def _att2_body(dst_n, src_n, rt_n, dst_h, src_h, ra_h, rb_h, bidx_h,
               psrc_hbm, prel_hbm, tdst_hbm,
               out_n, out_w, *rest):
    _att_body(1, dst_n, src_n, rt_n, dst_h, src_h, ra_h, rb_h, bidx_h,
              psrc_hbm, prel_hbm, tdst_hbm,
              out_n, out_w, *rest)


def _mesh():
    return plsc.VectorSubcoreMesh(core_axis_name="c", subcore_axis_name="s")


_OUT_TYPE = (jax.ShapeDtypeStruct((NC * NPAD, D), jnp.float32),
             jax.ShapeDtypeStruct((NC * NPAD, 16), jnp.float32))

_SCRATCH = (
    [pltpu.VMEM_SHARED((NPAD, D), jnp.float32),
     pltpu.VMEM_SHARED((NPAD, 16), jnp.float32)]
    + [pltpu.VMEM((BLK, CH), jnp.int32)] * 8
    + [pltpu.VMEM((CH,), jnp.int32)] * 2
    + [pltpu.VMEM((CH, DG), jnp.float32)] * 6
    + [pltpu.VMEM((CH, 16), jnp.float32)] * 2
    + [pltpu.VMEM((CH, D), jnp.float32)] * 2
    + [pltpu.VMEM((CH, 16), jnp.float32)] * 2
    + [pltpu.SemaphoreType.DMA] * 6
)

_att1 = pl.kernel(
    _att1_body, mesh=_mesh(),
    compiler_params=pltpu.CompilerParams(
        needs_layout_passes=False, use_tc_tiling_on_sc=False),
    out_type=_OUT_TYPE, scratch_types=_SCRATCH)

_att2 = pl.kernel(
    _att2_body, mesh=_mesh(),
    compiler_params=pltpu.CompilerParams(
        needs_layout_passes=False, use_tc_tiling_on_sc=False),
    out_type=_OUT_TYPE, scratch_types=_SCRATCH)


def _normalize_rows(x):
    n = jnp.linalg.norm(x, axis=1, keepdims=True)
    return x / jnp.maximum(n, 1e-12)


def _pad_to(x, n, value=0):
    return jnp.pad(x, [(0, n - x.shape[0])] + [(0, 0)] * (x.ndim - 1),
                   constant_values=value)


def _scal16(*cols):
    rows = cols[0].shape[0]
    out = jnp.zeros((rows, 16), jnp.float32)
    for i, c in enumerate(cols):
        out = out.at[:, i].set(c)
    return out


@jax.jit
def kernel(Corpus_, batch_inputs, edge_list, edge_type, train_indices_nhop,
           entity_embeddings, relation_embeddings, W_entities, W_spgat,
           a_heads, a2_heads, a_out, a2_out):
    f32 = jnp.float32
    ent = _normalize_rows(entity_embeddings)
    rel = _normalize_rows(relation_embeddings)
    rel_aug = jnp.concatenate([rel, jnp.zeros((RPAD - RN, 128), f32)], axis=0)

    i32 = jnp.int32
    dst_n = _pad_to(edge_list[0].astype(i32), EPAD, N).reshape(-1, CH)
    src_n = _pad_to(edge_list[1].astype(i32), EPAD, 0).reshape(-1, CH)
    rt_n = _pad_to(edge_type.astype(i32), EPAD, RN).reshape(-1, CH)
    tin = train_indices_nhop.astype(i32)
    dst_h = _pad_to(tin[:, 3], HPAD, N).reshape(-1, CH)
    src_h = _pad_to(tin[:, 0], HPAD, 0).reshape(-1, CH)
    ra_h = _pad_to(tin[:, 1], HPAD, RN).reshape(-1, CH)
    rb_h = _pad_to(tin[:, 2], HPAD, RN).reshape(-1, CH)
    bidx = batch_inputs[:, 2].astype(i32)

    p_dst = jnp.concatenate(
        [ent @ a_heads[0, :, :128].T, ent @ a_heads[1, :, :128].T], axis=1)
    p_src = jnp.concatenate(
        [ent @ a_heads[0, :, 128:256].T, ent @ a_heads[1, :, 128:256].T], axis=1)
    p_rel = jnp.concatenate(
        [rel_aug @ a_heads[0, :, 256:].T, rel_aug @ a_heads[1, :, 256:].T], axis=1)
    sd0 = p_dst[:, :64] @ a2_heads[0, 0]
    sd1 = p_dst[:, 64:] @ a2_heads[1, 0]
    ss0 = p_src[:, :64] @ a2_heads[0, 0]
    ss1 = p_src[:, 64:] @ a2_heads[1, 0]
    sr0 = p_rel[:, :64] @ a2_heads[0, 0]
    sr1 = p_rel[:, 64:] @ a2_heads[1, 0]
    tdst = _pad_to(_scal16(sd0, sd1), NPAD)
    psrc_t = jnp.concatenate([p_src, _scal16(ss0, ss1)], axis=1)
    prel_t = jnp.concatenate([p_rel, _scal16(sr0, sr1)], axis=1)

    acc_n, acc_w = _att1(dst_n, src_n, rt_n, dst_h, src_h, ra_h, rb_h,
                         psrc_t, prel_t, tdst)
    acc_n = acc_n[:NPAD] + acc_n[NPAD:]
    acc_w = acc_w[:NPAD] + acc_w[NPAD:]
    r0 = acc_w[:N, 0:1]
    r1 = acc_w[:N, 1:2]
    h0 = (p_dst[:, :64] * r0 + acc_n[:N, :64]) / jnp.where(r0 == 0.0, 1e-12, r0)
    h1 = (p_dst[:, 64:] * r1 + acc_n[:N, 64:]) / jnp.where(r1 == 0.0, 1e-12, r1)
    x = jnp.concatenate([jax.nn.elu(h0), jax.nn.elu(h1)], axis=1)

    out_relation_1 = rel @ W_spgat
    orel_aug = jnp.concatenate(
        [out_relation_1, jnp.zeros((RPAD - RN, 128), f32)], axis=0)
    q_dst = x @ a_out[:, :128].T
    q_src = x @ a_out[:, 128:256].T
    q_rel = orel_aug @ a_out[:, 256:].T
    s2d = q_dst @ a2_out[0]
    s2s = q_src @ a2_out[0]
    s2r = q_rel @ a2_out[0]
    tdst2 = _pad_to(_scal16(s2d), NPAD)
    qsrc_t = jnp.concatenate([q_src, _scal16(s2s)], axis=1)
    qrel_t = jnp.concatenate([q_rel, _scal16(s2r)], axis=1)

    acc2_n, acc2_w = _att2(dst_n, src_n, rt_n, dst_h, src_h, ra_h, rb_h, bidx,
                           qsrc_t, qrel_t, tdst2)
    acc2_n = acc2_n[:NPAD] + acc2_n[NPAD:]
    acc2_w = acc2_w[:NPAD] + acc2_w[NPAD:]
    r2 = acc2_w[:N, 0:1]
    h2 = (q_dst * r2 + acc2_n[:N]) / jnp.where(r2 == 0.0, 1e-12, r2)
    x2 = jax.nn.elu(h2)
    mask = (acc2_w[:N, 1:2] > 0.0).astype(f32)

    out_entity_1 = _normalize_rows(ent @ W_entities + mask * x2)
    return (out_entity_1, out_relation_1)
